# baseline (speedup 1.0000x reference)
import functools

import jax
import jax.numpy as jnp
from jax import lax
from jax.experimental import pallas as pl
from jax.experimental.pallas import tpu as pltpu
from jax.experimental.pallas import tpu_sc as plsc

N_NODES = 10000
D_FEAT = 256
D_LATENT = 128
N_EDGES = 160000

_NC = 2
_NS = 16
_NW = _NC * _NS
_L = 16

_EPW = N_EDGES // _NW
_CHUNK = 128
_NCH = -(-_EPW // _CHUNK)
_LAST = _EPW - _CHUNK
_NPAIR = _NCH // 2


def _encode_matmul(x, W):
    M, K = x.shape
    _, N = W.shape
    BM = 1024

    def body(x_ref, w_ref, z_ref, n_ref):
        z = jnp.dot(x_ref[...], w_ref[...], preferred_element_type=jnp.float32)
        z_ref[...] = z
        n_ref[...] = jnp.sum(z * z, axis=1)

    return pl.pallas_call(
        body,
        grid=(pl.cdiv(M, BM),),
        in_specs=[
            pl.BlockSpec((BM, K), lambda i: (i, 0)),
            pl.BlockSpec((K, N), lambda i: (0, 0)),
        ],
        out_specs=[
            pl.BlockSpec((BM, N), lambda i: (i, 0)),
            pl.BlockSpec((BM,), lambda i: (i,)),
        ],
        out_shape=[
            jax.ShapeDtypeStruct((M, N), jnp.float32),
            jax.ShapeDtypeStruct((M,), jnp.float32),
        ],
    )(x, W)


def _chunk_base(c):
    return jnp.minimum(c * _CHUNK, _LAST)


def _decode_body(z_hbm, n_hbm, src_hbm, dst_hbm, out_hbm,
                 idx_s, idx_d, buf0, buf1, n_local,
                 tbuf, out_v,
                 sem_r0, sem_a0, sem_r1, sem_a1):
    wid = lax.axis_index("s") * _NC + lax.axis_index("c")
    ebase = wid * _EPW

    pltpu.sync_copy(src_hbm.at[pl.ds(ebase, _EPW)], idx_s)
    pltpu.sync_copy(dst_hbm.at[pl.ds(ebase, _EPW)], idx_d)
    pltpu.sync_copy(n_hbm, n_local)

    def s_slice(c):
        return idx_s.at[pl.ds(_chunk_base(c), _CHUNK)]

    def d_slice(c):
        return idx_d.at[pl.ds(_chunk_base(c), _CHUNK)]

    def issue_src(c, buf, sem):
        pltpu.async_copy(z_hbm.at[s_slice(c)], buf, sem)

    def wait_src(buf, sem):
        pltpu.make_async_copy(z_hbm.at[s_slice(0)], buf, sem).wait()

    def issue_add(c, buf, sem_a):
        pltpu.async_copy(z_hbm.at[d_slice(c)], buf, sem_a, add=True)

    def wait_add(buf, sem_a):
        pltpu.make_async_copy(z_hbm.at[d_slice(0)], buf, sem_a).wait()

    def compute(c, buf):
        b = _chunk_base(c)

        def group_body(g, carry):
            for e in range(_L):
                row = g * _L + e
                w0 = buf[row, pl.ds(0, _L)]
                w4 = buf[row, pl.ds(4 * _L, _L)]
                accs = [w0 * w0, w4 * w4]
                for k in range(D_LATENT // _L):
                    if k % 4 != 0:
                        kk = (k % 4) // 2
                        wk = buf[row, pl.ds(k * _L, _L)]
                        accs[kk] = accs[kk] + wk * wk
                tbuf[pl.ds(e * 17, _L)] = accs[0] + accs[1]
            rowv = lax.iota(jnp.int32, _L) * 17
            accs = [
                plsc.load_gather(tbuf, [rowv]),
                plsc.load_gather(tbuf, [rowv + 1]),
            ]
            for k in range(2, _L):
                accs[k % 2] = accs[k % 2] + plsc.load_gather(tbuf, [rowv + k])
            nsv = plsc.load_gather(n_local, [idx_s[pl.ds(b + g * _L, _L)]])
            ndv = plsc.load_gather(n_local, [idx_d[pl.ds(b + g * _L, _L)]])
            acc = 0.5 * (accs[0] + accs[1] - nsv - ndv)
            ex = jnp.exp(-jnp.abs(acc))
            sig = jnp.where(acc >= 0.0, 1.0 / (1.0 + ex), ex / (1.0 + ex))
            out_v[pl.ds(b + g * _L, _L)] = sig
            return carry

        lax.fori_loop(0, _CHUNK // _L, group_body, 0)

    issue_src(0, buf0, sem_r0)
    wait_src(buf0, sem_r0)
    issue_add(0, buf0, sem_a0)
    issue_src(1, buf1, sem_r1)

    def pair_body(p, carry):
        c0 = 2 * p

        def step(c, buf, sem_r, sem_a, nbuf, nsem_r, nsem_a):
            @pl.when(c + 1 < _NCH)
            def _():
                wait_src(nbuf, nsem_r)
                issue_add(c + 1, nbuf, nsem_a)

            wait_add(buf, sem_a)
            compute(c, buf)

            @pl.when(c + 2 < _NCH)
            def _():
                issue_src(c + 2, buf, sem_r)

        step(c0, buf0, sem_r0, sem_a0, buf1, sem_r1, sem_a1)
        step(c0 + 1, buf1, sem_r1, sem_a1, buf0, sem_r0, sem_a0)
        return carry

    lax.fori_loop(0, _NPAIR, pair_body, 0)
    pltpu.sync_copy(out_v, out_hbm.at[pl.ds(ebase, _EPW)])


def _decode(z, n, src, dst):
    mesh = plsc.VectorSubcoreMesh(core_axis_name="c", subcore_axis_name="s")
    k = functools.partial(
        pl.kernel,
        out_type=jax.ShapeDtypeStruct((N_EDGES,), jnp.float32),
        mesh=mesh,
        scratch_types=[
            pltpu.VMEM((_EPW,), jnp.int32),
            pltpu.VMEM((_EPW,), jnp.int32),
            pltpu.VMEM((_CHUNK, D_LATENT), jnp.float32),
            pltpu.VMEM((_CHUNK, D_LATENT), jnp.float32),
            pltpu.VMEM((N_NODES,), jnp.float32),
            pltpu.VMEM((_L * 17,), jnp.float32),
            pltpu.VMEM((_EPW,), jnp.float32),
            pltpu.SemaphoreType.DMA,
            pltpu.SemaphoreType.DMA,
            pltpu.SemaphoreType.DMA,
            pltpu.SemaphoreType.DMA,
        ],
        compiler_params=pltpu.CompilerParams(needs_layout_passes=False),
    )(_decode_body)
    return k(z, n, src, dst)


def kernel(x, edge_index, W):
    z, n = _encode_matmul(x, W)
    ei = edge_index.astype(jnp.int32)
    return _decode(z, n, ei[0], ei[1])


# speedup vs baseline: 1.0952x; 1.0952x over previous
import functools

import jax
import jax.numpy as jnp
from jax import lax
from jax.experimental import pallas as pl
from jax.experimental.pallas import tpu as pltpu
from jax.experimental.pallas import tpu_sc as plsc

N_NODES = 10000
D_FEAT = 256
D_LATENT = 128
N_EDGES = 160000

_NC = 2
_NS = 16
_NW = _NC * _NS
_L = 16

_EPW = N_EDGES // _NW
_CHUNK = 128
_NCH = -(-_EPW // _CHUNK)
_LAST = _EPW - _CHUNK
_NPAIR = _NCH // 2


def _encode_matmul(x, W):
    M, K = x.shape
    _, N = W.shape
    BM = 2000

    def body(x_ref, w_ref, z_ref):
        z_ref[...] = jnp.dot(x_ref[...], w_ref[...],
                             preferred_element_type=jnp.float32)

    return pl.pallas_call(
        body,
        grid=(M // BM,),
        in_specs=[
            pl.BlockSpec((BM, K), lambda i: (i, 0)),
            pl.BlockSpec((K, N), lambda i: (0, 0)),
        ],
        out_specs=pl.BlockSpec((BM, N), lambda i: (i, 0)),
        out_shape=jax.ShapeDtypeStruct((M, N), jnp.float32),
    )(x, W)


def _last_lane_mask():
    return lax.iota(jnp.int32, _L) == (_L - 1)


def _chunk_base(c):
    return jnp.minimum(c * _CHUNK, _LAST)


def _decode_body(z_hbm, ei_hbm, out_hbm,
                 idx_s, idx_d, rows_s0, rows_d0, rows_s1, rows_d1,
                 tbuf, out_v,
                 sem_s0, sem_d0, sem_s1, sem_d1):
    wid = lax.axis_index("s") * _NC + lax.axis_index("c")
    ebase = wid * _EPW

    pltpu.sync_copy(ei_hbm.at[pl.ds(ebase, _EPW)], idx_s)
    pltpu.sync_copy(ei_hbm.at[pl.ds(N_EDGES + ebase, _EPW)], idx_d)

    def issue(c, rs, rd, ss, sd):
        b = _chunk_base(c)
        pltpu.async_copy(z_hbm.at[idx_s.at[pl.ds(b, _CHUNK)]], rs, ss)
        pltpu.async_copy(z_hbm.at[idx_d.at[pl.ds(b, _CHUNK)]], rd, sd)

    def wait(rs, rd, ss, sd):
        pltpu.make_async_copy(z_hbm.at[idx_s.at[pl.ds(0, _CHUNK)]],
                              rs, ss).wait()
        pltpu.make_async_copy(z_hbm.at[idx_d.at[pl.ds(0, _CHUNK)]],
                              rd, sd).wait()

    def compute(c, rows_s, rows_d):
        b = _chunk_base(c)

        def group_body(g, carry):
            for e in range(_L):
                row = g * _L + e
                accs = [
                    rows_s[row, pl.ds(4 * k * _L, _L)]
                    * rows_d[row, pl.ds(4 * k * _L, _L)]
                    for k in range(2)
                ]
                for k in range(D_LATENT // _L):
                    if k % 4 != 0:
                        kk = (k % 4) // 2
                        accs[kk] = accs[kk] + (
                            rows_s[row, pl.ds(k * _L, _L)]
                            * rows_d[row, pl.ds(k * _L, _L)]
                        )
                cs = jnp.cumsum(accs[0] + accs[1])
                plsc.store_scatter(tbuf, [jnp.full((_L,), e, jnp.int32)],
                                   cs, mask=_last_lane_mask())
            acc = tbuf[pl.ds(0, _L)]
            ex = jnp.exp(-jnp.abs(acc))
            sig = jnp.where(acc >= 0.0, 1.0 / (1.0 + ex), ex / (1.0 + ex))
            out_v[pl.ds(b + g * _L, _L)] = sig
            return carry

        lax.fori_loop(0, _CHUNK // _L, group_body, 0)

    issue(0, rows_s0, rows_d0, sem_s0, sem_d0)

    def pair_body(p, carry):
        c0 = 2 * p
        issue(c0 + 1, rows_s1, rows_d1, sem_s1, sem_d1)
        wait(rows_s0, rows_d0, sem_s0, sem_d0)
        compute(c0, rows_s0, rows_d0)

        @pl.when(p < _NPAIR - 1)
        def _():
            issue(c0 + 2, rows_s0, rows_d0, sem_s0, sem_d0)

        wait(rows_s1, rows_d1, sem_s1, sem_d1)
        compute(c0 + 1, rows_s1, rows_d1)
        return carry

    lax.fori_loop(0, _NPAIR, pair_body, 0)
    pltpu.sync_copy(out_v, out_hbm.at[pl.ds(ebase, _EPW)])


def _decode(z, ei):
    mesh = plsc.VectorSubcoreMesh(core_axis_name="c", subcore_axis_name="s")
    k = functools.partial(
        pl.kernel,
        out_type=jax.ShapeDtypeStruct((N_EDGES,), jnp.float32),
        mesh=mesh,
        scratch_types=[
            pltpu.VMEM((_EPW,), jnp.int32),
            pltpu.VMEM((_EPW,), jnp.int32),
            pltpu.VMEM((_CHUNK, D_LATENT), jnp.float32),
            pltpu.VMEM((_CHUNK, D_LATENT), jnp.float32),
            pltpu.VMEM((_CHUNK, D_LATENT), jnp.float32),
            pltpu.VMEM((_CHUNK, D_LATENT), jnp.float32),
            pltpu.VMEM((_L,), jnp.float32),
            pltpu.VMEM((_EPW,), jnp.float32),
            pltpu.SemaphoreType.DMA,
            pltpu.SemaphoreType.DMA,
            pltpu.SemaphoreType.DMA,
            pltpu.SemaphoreType.DMA,
        ],
        compiler_params=pltpu.CompilerParams(needs_layout_passes=False),
    )(_decode_body)
    return k(z, ei)


def kernel(x, edge_index, W):
    z = _encode_matmul(x, W)
    return _decode(z, edge_index.astype(jnp.int32).reshape(-1))


# speedup vs baseline: 1.3353x; 1.2193x over previous
import functools

import jax
import jax.numpy as jnp
from jax import lax
from jax.experimental import pallas as pl
from jax.experimental.pallas import tpu as pltpu
from jax.experimental.pallas import tpu_sc as plsc

N_NODES = 10000
D_FEAT = 256
D_LATENT = 128
N_EDGES = 160000

_NC = 2
_NS = 16
_NW = _NC * _NS
_L = 16

_EPW = N_EDGES // _NW
_CHUNK = 128
_NCH = -(-_EPW // _CHUNK)
_LAST = _EPW - _CHUNK
_NPAIR = _NCH // 2


def _encode_matmul(x, W):
    M, K = x.shape
    _, N = W.shape
    BM = 2000

    def body(x_ref, w_ref, z_ref):
        z_ref[...] = jnp.dot(x_ref[...], w_ref[...],
                             preferred_element_type=jnp.float32)

    return pl.pallas_call(
        body,
        grid=(M // BM,),
        in_specs=[
            pl.BlockSpec((BM, K), lambda i: (i, 0)),
            pl.BlockSpec((K, N), lambda i: (0, 0)),
        ],
        out_specs=pl.BlockSpec((BM, N), lambda i: (i, 0)),
        out_shape=jax.ShapeDtypeStruct((M, N), jnp.float32),
    )(x, W)


def _last_lane_mask():
    return lax.iota(jnp.int32, _L) == (_L - 1)


def _chunk_base(c):
    return jnp.minimum(c * _CHUNK, _LAST)


def _decode_body(z_hbm, ei_hbm, out_hbm,
                 idx_s, idx_d, rows_s0, rows_d0, rows_s1, rows_d1,
                 tbuf, out_v,
                 sem_s0, sem_d0, sem_s1, sem_d1):
    wid = lax.axis_index("s") * _NC + lax.axis_index("c")
    ebase = wid * _EPW

    pltpu.sync_copy(ei_hbm.at[pl.ds(ebase, _EPW)], idx_s)
    pltpu.sync_copy(ei_hbm.at[pl.ds(N_EDGES + ebase, _EPW)], idx_d)

    def issue(c, rs, rd, ss, sd):
        b = _chunk_base(c)
        pltpu.async_copy(z_hbm.at[idx_s.at[pl.ds(b, _CHUNK)]], rs, ss)
        pltpu.async_copy(z_hbm.at[idx_d.at[pl.ds(b, _CHUNK)]], rd, sd)

    def wait(rs, rd, ss, sd):
        pltpu.make_async_copy(z_hbm.at[idx_s.at[pl.ds(0, _CHUNK)]],
                              rs, ss).wait()
        pltpu.make_async_copy(z_hbm.at[idx_d.at[pl.ds(0, _CHUNK)]],
                              rd, sd).wait()

    def compute(c, rows_s, rows_d):
        b = _chunk_base(c)

        def group_body(g, carry):
            for e in range(_L):
                row = g * _L + e
                accs = [
                    rows_s[row, pl.ds(4 * k * _L, _L)]
                    * rows_d[row, pl.ds(4 * k * _L, _L)]
                    for k in range(2)
                ]
                for k in range(D_LATENT // _L):
                    if k % 4 != 0:
                        kk = (k % 4) // 2
                        accs[kk] = accs[kk] + (
                            rows_s[row, pl.ds(k * _L, _L)]
                            * rows_d[row, pl.ds(k * _L, _L)]
                        )
                tbuf[pl.ds(e * 17, _L)] = accs[0] + accs[1]
            rowv = lax.iota(jnp.int32, _L) * 17
            raccs = [
                plsc.load_gather(tbuf, [rowv]),
                plsc.load_gather(tbuf, [rowv + 1]),
            ]
            for k in range(2, _L):
                raccs[k % 2] = raccs[k % 2] + plsc.load_gather(tbuf, [rowv + k])
            acc = raccs[0] + raccs[1]
            ex = jnp.exp(-jnp.abs(acc))
            sig = jnp.where(acc >= 0.0, 1.0 / (1.0 + ex), ex / (1.0 + ex))
            out_v[pl.ds(b + g * _L, _L)] = sig
            return carry

        lax.fori_loop(0, _CHUNK // _L, group_body, 0)

    issue(0, rows_s0, rows_d0, sem_s0, sem_d0)

    def pair_body(p, carry):
        c0 = 2 * p
        issue(c0 + 1, rows_s1, rows_d1, sem_s1, sem_d1)
        wait(rows_s0, rows_d0, sem_s0, sem_d0)
        compute(c0, rows_s0, rows_d0)

        @pl.when(p < _NPAIR - 1)
        def _():
            issue(c0 + 2, rows_s0, rows_d0, sem_s0, sem_d0)

        wait(rows_s1, rows_d1, sem_s1, sem_d1)
        compute(c0 + 1, rows_s1, rows_d1)
        return carry

    lax.fori_loop(0, _NPAIR, pair_body, 0)
    pltpu.sync_copy(out_v, out_hbm.at[pl.ds(ebase, _EPW)])


def _decode(z, ei):
    mesh = plsc.VectorSubcoreMesh(core_axis_name="c", subcore_axis_name="s")
    k = functools.partial(
        pl.kernel,
        out_type=jax.ShapeDtypeStruct((N_EDGES,), jnp.float32),
        mesh=mesh,
        scratch_types=[
            pltpu.VMEM((_EPW,), jnp.int32),
            pltpu.VMEM((_EPW,), jnp.int32),
            pltpu.VMEM((_CHUNK, D_LATENT), jnp.float32),
            pltpu.VMEM((_CHUNK, D_LATENT), jnp.float32),
            pltpu.VMEM((_CHUNK, D_LATENT), jnp.float32),
            pltpu.VMEM((_CHUNK, D_LATENT), jnp.float32),
            pltpu.VMEM((_L * 17,), jnp.float32),
            pltpu.VMEM((_EPW,), jnp.float32),
            pltpu.SemaphoreType.DMA,
            pltpu.SemaphoreType.DMA,
            pltpu.SemaphoreType.DMA,
            pltpu.SemaphoreType.DMA,
        ],
        compiler_params=pltpu.CompilerParams(needs_layout_passes=False),
    )(_decode_body)
    return k(z, ei)


def kernel(x, edge_index, W):
    z = _encode_matmul(x, W)
    return _decode(z, edge_index.astype(jnp.int32).reshape(-1))


# speedup vs baseline: 1.4358x; 1.0752x over previous
import functools

import jax
import jax.numpy as jnp
from jax import lax
from jax.experimental import pallas as pl
from jax.experimental.pallas import tpu as pltpu
from jax.experimental.pallas import tpu_sc as plsc

N_NODES = 10000
D_FEAT = 256
D_LATENT = 128
N_EDGES = 160000

_NC = 2
_NS = 16
_NW = _NC * _NS
_L = 16

_EPW = N_EDGES // _NW
_CHUNK = 128
_NCH = -(-_EPW // _CHUNK)
_LAST = _EPW - _CHUNK
_NPAIR = _NCH // 2


def _encode_matmul(x, W):
    M, K = x.shape
    _, N = W.shape
    BM = 2000

    def body(x_ref, w_ref, z_ref):
        z = jnp.dot(x_ref[...], w_ref[...],
                    preferred_element_type=jnp.float32)
        zb = z.astype(jnp.bfloat16)
        lo = lax.bitcast_convert_type(zb[:, 0:N // 2], jnp.uint16)
        hi = lax.bitcast_convert_type(zb[:, N // 2:N], jnp.uint16)
        zi = lo.astype(jnp.uint32) | (hi.astype(jnp.uint32) << 16)
        z_ref[:, 0:N // 2] = lax.bitcast_convert_type(zi, jnp.int32)

    return pl.pallas_call(
        body,
        grid=(M // BM,),
        in_specs=[
            pl.BlockSpec((BM, K), lambda i: (i, 0)),
            pl.BlockSpec((K, N), lambda i: (0, 0)),
        ],
        out_specs=pl.BlockSpec((BM, N), lambda i: (i, 0)),
        out_shape=jax.ShapeDtypeStruct((M, N), jnp.int32),
    )(x, W)


def _last_lane_mask():
    return lax.iota(jnp.int32, _L) == (_L - 1)


def _chunk_base(c):
    return jnp.minimum(c * _CHUNK, _LAST)


def _decode_body(z_hbm, ei_hbm, out_hbm,
                 idx_s, idx_d, rows_s0, rows_d0, rows_s1, rows_d1,
                 tbuf, out_v,
                 sem_s0, sem_d0, sem_s1, sem_d1):
    wid = lax.axis_index("s") * _NC + lax.axis_index("c")
    ebase = wid * _EPW

    pltpu.sync_copy(ei_hbm.at[pl.ds(ebase, _EPW)], idx_s)
    pltpu.sync_copy(ei_hbm.at[pl.ds(N_EDGES + ebase, _EPW)], idx_d)

    def issue(c, rs, rd, ss, sd):
        b = _chunk_base(c)
        pltpu.async_copy(z_hbm.at[idx_s.at[pl.ds(b, _CHUNK)]], rs, ss)
        pltpu.async_copy(z_hbm.at[idx_d.at[pl.ds(b, _CHUNK)]], rd, sd)

    def wait(rs, rd, ss, sd):
        pltpu.make_async_copy(z_hbm.at[idx_s.at[pl.ds(0, _CHUNK)]],
                              rs, ss).wait()
        pltpu.make_async_copy(z_hbm.at[idx_d.at[pl.ds(0, _CHUNK)]],
                              rd, sd).wait()

    def compute(c, rows_s, rows_d):
        b = _chunk_base(c)

        def group_body(g, carry):
            for e in range(_L):
                row = g * _L + e
                accs = [None, None]
                for k in range(D_LATENT // (2 * _L)):
                    sv = plsc.bitcast(rows_s[row, pl.ds(k * _L, _L)],
                                      jnp.bfloat16)
                    dv = plsc.bitcast(rows_d[row, pl.ds(k * _L, _L)],
                                      jnp.bfloat16)
                    sa, sb = plsc.unpack(sv, format=plsc.PackFormat.INTERLEAVED,
                                         preferred_element_type=jnp.float32)
                    da, db = plsc.unpack(dv, format=plsc.PackFormat.INTERLEAVED,
                                         preferred_element_type=jnp.float32)
                    for kk, (u, v) in enumerate(((sa, da), (sb, db))):
                        p = u * v
                        accs[kk] = p if accs[kk] is None else accs[kk] + p
                tbuf[pl.ds(e * 17, _L)] = accs[0] + accs[1]
            rowv = lax.iota(jnp.int32, _L) * 17
            raccs = [
                plsc.load_gather(tbuf, [rowv]),
                plsc.load_gather(tbuf, [rowv + 1]),
            ]
            for k in range(2, _L):
                raccs[k % 2] = raccs[k % 2] + plsc.load_gather(tbuf, [rowv + k])
            acc = raccs[0] + raccs[1]
            ex = jnp.exp(-jnp.abs(acc))
            sig = jnp.where(acc >= 0.0, 1.0 / (1.0 + ex), ex / (1.0 + ex))
            out_v[pl.ds(b + g * _L, _L)] = sig
            return carry

        lax.fori_loop(0, _CHUNK // _L, group_body, 0)

    issue(0, rows_s0, rows_d0, sem_s0, sem_d0)

    def pair_body(p, carry):
        c0 = 2 * p
        issue(c0 + 1, rows_s1, rows_d1, sem_s1, sem_d1)
        wait(rows_s0, rows_d0, sem_s0, sem_d0)
        compute(c0, rows_s0, rows_d0)

        @pl.when(p < _NPAIR - 1)
        def _():
            issue(c0 + 2, rows_s0, rows_d0, sem_s0, sem_d0)

        wait(rows_s1, rows_d1, sem_s1, sem_d1)
        compute(c0 + 1, rows_s1, rows_d1)
        return carry

    lax.fori_loop(0, _NPAIR, pair_body, 0)
    pltpu.sync_copy(out_v, out_hbm.at[pl.ds(ebase, _EPW)])


def _decode(z, ei):
    mesh = plsc.VectorSubcoreMesh(core_axis_name="c", subcore_axis_name="s")
    k = functools.partial(
        pl.kernel,
        out_type=jax.ShapeDtypeStruct((N_EDGES,), jnp.float32),
        mesh=mesh,
        scratch_types=[
            pltpu.VMEM((_EPW,), jnp.int32),
            pltpu.VMEM((_EPW,), jnp.int32),
            pltpu.VMEM((_CHUNK, D_LATENT), jnp.int32),
            pltpu.VMEM((_CHUNK, D_LATENT), jnp.int32),
            pltpu.VMEM((_CHUNK, D_LATENT), jnp.int32),
            pltpu.VMEM((_CHUNK, D_LATENT), jnp.int32),
            pltpu.VMEM((_L * 17,), jnp.float32),
            pltpu.VMEM((_EPW,), jnp.float32),
            pltpu.SemaphoreType.DMA,
            pltpu.SemaphoreType.DMA,
            pltpu.SemaphoreType.DMA,
            pltpu.SemaphoreType.DMA,
        ],
        compiler_params=pltpu.CompilerParams(needs_layout_passes=False),
    )(_decode_body)
    return k(z, ei)


def kernel(x, edge_index, W):
    z32 = _encode_matmul(x, W)
    return _decode(z32, edge_index.astype(jnp.int32).reshape(-1))


# speedup vs baseline: 1.4740x; 1.0266x over previous
import functools

import jax
import jax.numpy as jnp
from jax import lax
from jax.experimental import pallas as pl
from jax.experimental.pallas import tpu as pltpu
from jax.experimental.pallas import tpu_sc as plsc

N_NODES = 10000
D_FEAT = 256
D_LATENT = 128
N_EDGES = 160000

_NC = 2
_NS = 16
_NW = _NC * _NS
_L = 16

_EPW = N_EDGES // _NW
_CHUNK = 128
_NCH = -(-_EPW // _CHUNK)
_LAST = _EPW - _CHUNK
_NPAIR = _NCH // 2


def _encode_matmul(x, W):
    M, K = x.shape
    _, N = W.shape
    BM = 2000

    def body(x_ref, w_ref, z_ref):
        z = jnp.dot(x_ref[...], w_ref[...],
                    preferred_element_type=jnp.float32)
        zb = z.astype(jnp.bfloat16)
        lo = lax.bitcast_convert_type(zb[:, 0:N // 2], jnp.uint16)
        hi = lax.bitcast_convert_type(zb[:, N // 2:N], jnp.uint16)
        zi = lo.astype(jnp.uint32) | (hi.astype(jnp.uint32) << 16)
        z_ref[:, 0:N // 2] = lax.bitcast_convert_type(zi, jnp.int32)

    return pl.pallas_call(
        body,
        grid=(M // BM,),
        in_specs=[
            pl.BlockSpec((BM, K), lambda i: (i, 0)),
            pl.BlockSpec((K, N), lambda i: (0, 0)),
        ],
        out_specs=pl.BlockSpec((BM, N), lambda i: (i, 0)),
        out_shape=jax.ShapeDtypeStruct((M, N), jnp.int32),
    )(x, W)


def _last_lane_mask():
    return lax.iota(jnp.int32, _L) == (_L - 1)


def _chunk_base(c):
    return jnp.minimum(c * _CHUNK, _LAST)


def _decode_body(z_hbm, ei_hbm, out_hbm,
                 idx_s, idx_d, rows_s0, rows_d0, rows_s1, rows_d1,
                 tbuf, out_v,
                 sem_s0, sem_d0, sem_s1, sem_d1):
    wid = lax.axis_index("s") * _NC + lax.axis_index("c")
    ebase = wid * _EPW

    pltpu.sync_copy(ei_hbm.at[pl.ds(ebase, _EPW)], idx_s)
    pltpu.sync_copy(ei_hbm.at[pl.ds(N_EDGES + ebase, _EPW)], idx_d)

    def issue(c, rs, rd, ss, sd):
        b = _chunk_base(c)
        pltpu.async_copy(z_hbm.at[idx_s.at[pl.ds(b, _CHUNK)]], rs, ss)
        pltpu.async_copy(z_hbm.at[idx_d.at[pl.ds(b, _CHUNK)]], rd, sd)

    def wait(rs, rd, ss, sd):
        pltpu.make_async_copy(z_hbm.at[idx_s.at[pl.ds(0, _CHUNK)]],
                              rs, ss).wait()
        pltpu.make_async_copy(z_hbm.at[idx_d.at[pl.ds(0, _CHUNK)]],
                              rd, sd).wait()

    def compute(c, rows_s, rows_d):
        b = _chunk_base(c)

        def group_body(g, carry):
            for e in range(_L):
                row = g * _L + e
                accs = [None, None]
                for k in range(D_LATENT // (2 * _L)):
                    sv = plsc.bitcast(rows_s[row, pl.ds(k * _L, _L)],
                                      jnp.bfloat16)
                    dv = plsc.bitcast(rows_d[row, pl.ds(k * _L, _L)],
                                      jnp.bfloat16)
                    pa, pb = plsc.unpack(sv * dv,
                                         format=plsc.PackFormat.INTERLEAVED,
                                         preferred_element_type=jnp.float32)
                    for kk, p in enumerate((pa, pb)):
                        accs[kk] = p if accs[kk] is None else accs[kk] + p
                tbuf[pl.ds(e * 17, _L)] = accs[0] + accs[1]
            rowv = lax.iota(jnp.int32, _L) * 17
            raccs = [
                plsc.load_gather(tbuf, [rowv]),
                plsc.load_gather(tbuf, [rowv + 1]),
            ]
            for k in range(2, _L):
                raccs[k % 2] = raccs[k % 2] + plsc.load_gather(tbuf, [rowv + k])
            acc = raccs[0] + raccs[1]
            ex = jnp.exp(-jnp.abs(acc))
            sig = jnp.where(acc >= 0.0, 1.0 / (1.0 + ex), ex / (1.0 + ex))
            out_v[pl.ds(b + g * _L, _L)] = sig
            return carry

        lax.fori_loop(0, _CHUNK // _L, group_body, 0)

    issue(0, rows_s0, rows_d0, sem_s0, sem_d0)

    def pair_body(p, carry):
        c0 = 2 * p
        issue(c0 + 1, rows_s1, rows_d1, sem_s1, sem_d1)
        wait(rows_s0, rows_d0, sem_s0, sem_d0)
        compute(c0, rows_s0, rows_d0)

        @pl.when(p < _NPAIR - 1)
        def _():
            issue(c0 + 2, rows_s0, rows_d0, sem_s0, sem_d0)

        wait(rows_s1, rows_d1, sem_s1, sem_d1)
        compute(c0 + 1, rows_s1, rows_d1)
        return carry

    lax.fori_loop(0, _NPAIR, pair_body, 0)
    pltpu.sync_copy(out_v, out_hbm.at[pl.ds(ebase, _EPW)])


def _decode(z, ei):
    mesh = plsc.VectorSubcoreMesh(core_axis_name="c", subcore_axis_name="s")
    k = functools.partial(
        pl.kernel,
        out_type=jax.ShapeDtypeStruct((N_EDGES,), jnp.float32),
        mesh=mesh,
        scratch_types=[
            pltpu.VMEM((_EPW,), jnp.int32),
            pltpu.VMEM((_EPW,), jnp.int32),
            pltpu.VMEM((_CHUNK, D_LATENT), jnp.int32),
            pltpu.VMEM((_CHUNK, D_LATENT), jnp.int32),
            pltpu.VMEM((_CHUNK, D_LATENT), jnp.int32),
            pltpu.VMEM((_CHUNK, D_LATENT), jnp.int32),
            pltpu.VMEM((_L * 17,), jnp.float32),
            pltpu.VMEM((_EPW,), jnp.float32),
            pltpu.SemaphoreType.DMA,
            pltpu.SemaphoreType.DMA,
            pltpu.SemaphoreType.DMA,
            pltpu.SemaphoreType.DMA,
        ],
        compiler_params=pltpu.CompilerParams(needs_layout_passes=False),
    )(_decode_body)
    return k(z, ei)


def kernel(x, edge_index, W):
    z32 = _encode_matmul(x, W)
    return _decode(z32, edge_index.astype(jnp.int32).reshape(-1))
